# Initial kernel scaffold; baseline (speedup 1.0000x reference)
import jax
import jax.numpy as jnp
from jax.experimental import pallas as pl


def kernel(skills, ability_table, difficulty_table, discrimination_table):
    raise NotImplementedError("write your pallas kernel here")


# speedup vs baseline: 1.2378x; 1.2378x over previous
import functools

import jax
import jax.numpy as jnp
from jax import lax
from jax.experimental import pallas as pl
from jax.experimental.pallas import tpu as pltpu
from jax.experimental.pallas import tpu_sc as plsc

_NC = 2
_NS = 16
_NW = _NC * _NS
_LANES = 16
_CHUNK = 128


@functools.partial(jax.jit, static_argnames=("batch",))
def _irt_sc(skills, ability16, difficulty, discrimination, *, batch):
    b_per_w = batch // _NW
    n_chunks = b_per_w // _CHUNK
    mesh = plsc.VectorSubcoreMesh(
        core_axis_name="c", subcore_axis_name="s",
        num_cores=_NC, num_subcores=_NS)

    @functools.partial(
        pl.kernel,
        out_type=jax.ShapeDtypeStruct((_NW, n_chunks, _CHUNK), jnp.float32),
        mesh=mesh,
        scratch_types=[
            pltpu.VMEM((n_chunks, _CHUNK), jnp.int32),
            pltpu.VMEM((n_chunks, _CHUNK), jnp.float32),
            pltpu.VMEM((n_chunks, _CHUNK), jnp.float32),
            pltpu.VMEM((_LANES,), jnp.float32),
            pltpu.SemaphoreType.DMA,
        ],
    )
    def k(skills_hbm, ab_hbm, diff_hbm, disc_hbm, out_hbm,
          idx_v, diff_v, disc_v, ab_v, sem):
        wid = lax.axis_index("s") * _NC + lax.axis_index("c")
        pltpu.sync_copy(skills_hbm.at[wid], idx_v)
        pltpu.sync_copy(ab_hbm, ab_v)
        copies = []
        for j in range(n_chunks):
            copies.append(
                pltpu.async_copy(diff_hbm.at[idx_v.at[j]], diff_v.at[j], sem))
            copies.append(
                pltpu.async_copy(disc_hbm.at[idx_v.at[j]], disc_v.at[j], sem))
        for cp in copies:
            cp.wait()
        a = ab_v[:]
        for j in range(n_chunks):
            for i in range(_CHUNK // _LANES):
                sl = pl.ds(i * _LANES, _LANES)
                x = disc_v[j, sl] * (a - diff_v[j, sl])
                out_hbm_val = 1.0 / (1.0 + jnp.exp(-x))
                diff_v[j, sl] = out_hbm_val
        pltpu.sync_copy(diff_v, out_hbm.at[wid])

    return k(skills, ability16, difficulty, discrimination)


def kernel(skills, ability_table, difficulty_table, discrimination_table):
    batch = skills.shape[0]
    skills3 = skills.astype(jnp.int32).reshape(_NW, batch // _NW // _CHUNK, _CHUNK)
    ability16 = jnp.broadcast_to(ability_table.reshape(()), (_LANES,))
    diff = difficulty_table.reshape(-1)
    disc = discrimination_table.reshape(-1)
    out = _irt_sc(skills3, ability16, diff, disc, batch=batch)
    return out.reshape(batch, 1)


# speedup vs baseline: 1.2923x; 1.0441x over previous
import functools

import jax
import jax.numpy as jnp
from jax import lax
from jax.experimental import pallas as pl
from jax.experimental.pallas import tpu as pltpu
from jax.experimental.pallas import tpu_sc as plsc

_NC = 2
_NS = 16
_NW = _NC * _NS
_LANES = 16
_CHUNK = 128


@functools.partial(jax.jit, static_argnames=("batch",))
def _irt_sc(skills, ability16, difficulty, discrimination, *, batch):
    b_per_w = batch // _NW
    n_chunks = b_per_w // _CHUNK
    mesh = plsc.VectorSubcoreMesh(
        core_axis_name="c", subcore_axis_name="s",
        num_cores=_NC, num_subcores=_NS)

    @functools.partial(
        pl.kernel,
        out_type=jax.ShapeDtypeStruct((_NW, n_chunks, _CHUNK), jnp.float32),
        mesh=mesh,
        scratch_types=[
            pltpu.VMEM((n_chunks, _CHUNK), jnp.int32),
            pltpu.VMEM((n_chunks, _CHUNK), jnp.float32),
            pltpu.VMEM((n_chunks, _CHUNK), jnp.float32),
            pltpu.VMEM((_LANES,), jnp.float32),
            pltpu.SemaphoreType.DMA,
        ],
    )
    def k(skills_hbm, ab_hbm, diff_hbm, disc_hbm, out_hbm,
          idx_v, diff_v, disc_v, ab_v, sem):
        wid = lax.axis_index("s") * _NC + lax.axis_index("c")
        pltpu.sync_copy(skills_hbm.at[wid], idx_v)
        copies = []
        for j in range(n_chunks):
            copies.append(
                pltpu.async_copy(diff_hbm.at[idx_v.at[j]], diff_v.at[j], sem))
            copies.append(
                pltpu.async_copy(disc_hbm.at[idx_v.at[j]], disc_v.at[j], sem))
        pltpu.sync_copy(ab_hbm, ab_v)
        for cp in copies:
            cp.wait()
        a = ab_v[:]
        for j in range(n_chunks):
            for i in range(_CHUNK // _LANES):
                sl = pl.ds(i * _LANES, _LANES)
                x = disc_v[j, sl] * (a - diff_v[j, sl])
                out_hbm_val = 1.0 / (1.0 + jnp.exp(-x))
                diff_v[j, sl] = out_hbm_val
        pltpu.sync_copy(diff_v, out_hbm.at[wid])

    return k(skills, ability16, difficulty, discrimination)


def kernel(skills, ability_table, difficulty_table, discrimination_table):
    batch = skills.shape[0]
    skills3 = skills.astype(jnp.int32).reshape(_NW, batch // _NW // _CHUNK, _CHUNK)
    ability16 = jnp.broadcast_to(ability_table.reshape(()), (_LANES,))
    diff = difficulty_table.reshape(-1)
    disc = discrimination_table.reshape(-1)
    out = _irt_sc(skills3, ability16, diff, disc, batch=batch)
    return out.reshape(batch, 1)
